# Initial kernel scaffold; baseline (speedup 1.0000x reference)
import jax
import jax.numpy as jnp
from jax.experimental import pallas as pl


def kernel(h_idx, r_idx, edge_index, edge_rel, entity_emb):
    raise NotImplementedError("write your pallas kernel here")


# speedup vs baseline: 5.2605x; 5.2605x over previous
import functools

import jax
import jax.numpy as jnp
from jax import lax
from jax.experimental import pallas as pl
from jax.experimental.pallas import tpu as pltpu
from jax.experimental.pallas import tpu_sc as plsc

N = 10000
R = 8
D = 128
E = 320000
B = 1024

NC = 2
NS = 16
NW = NC * NS
CHUNK = 16
NPASS = D // CHUNK
SROWS = R * N
ACC_ROWS = SROWS
ENC_ROWS = SROWS + 8
TILE_ROWS = ACC_ROWS // NS
ZROWS = TILE_ROWS // 8

EBATCH = 128
EP = 327680
EROWS = EP // EBATCH
TBATCH = EROWS // NW
CCHUNK = 16
NCHUNK = TBATCH // CCHUNK
EPS = 1e-12


def _norm_body(x_ref, o_ref):
    x = x_ref[...]
    n = jnp.sqrt(jnp.sum(x * x, axis=1, keepdims=True))
    o_ref[...] = x / jnp.maximum(n, EPS)


def _normalize(emb):
    return pl.pallas_call(
        _norm_body,
        grid=(10,),
        in_specs=[pl.BlockSpec((N // 10, D), lambda i: (i, 0))],
        out_specs=pl.BlockSpec((N // 10, D), lambda i: (i, 0)),
        out_shape=jax.ShapeDtypeStruct((N, D), jnp.float32),
    )(emb)


def _idx_body(src_ref, rel_ref, dst_ref, sidx_ref, dstc_ref):
    c = pl.program_id(0)

    @pl.when(c == 0)
    def _():
        sidx_ref[...] = rel_ref[...] * N + src_ref[...]

    dstc_ref[...] = (dst_ref[...] + c * N)[None]


def _make_indices(src2, rel2, dst2):
    rows = E // 128
    return pl.pallas_call(
        _idx_body,
        grid=(NPASS,),
        in_specs=[
            pl.BlockSpec((rows, 128), lambda c: (0, 0)),
            pl.BlockSpec((rows, 128), lambda c: (0, 0)),
            pl.BlockSpec((rows, 128), lambda c: (0, 0)),
        ],
        out_specs=[
            pl.BlockSpec((rows, 128), lambda c: (0, 0)),
            pl.BlockSpec((1, rows, 128), lambda c: (c, 0, 0)),
        ],
        out_shape=[
            jax.ShapeDtypeStruct((rows, 128), jnp.int32),
            jax.ShapeDtypeStruct((NPASS, rows, 128), jnp.int32),
        ],
    )(src2, rel2, dst2)


def _sc_body(enc_hbm, dstc_hbm, sidx_hbm, hidx_hbm, en_hbm,
             acc_out, hrows_out,
             acc, dbuf, sbuf, rows, zbuf, hidx, hrow, gsem):
    cid = lax.axis_index("c")
    sid = lax.axis_index("s")
    wid = cid * NS + sid

    hper = B // NW
    pltpu.sync_copy(hidx_hbm.at[pl.ds(wid * hper, hper)], hidx)
    pltpu.async_copy(en_hbm.at[hidx], hrow, gsem).wait()
    pltpu.sync_copy(hrow, hrows_out.at[pl.ds(wid * hper, hper)])

    def _z(i, _):
        zbuf[i] = jnp.zeros((16,), jnp.float32)
        return 0
    lax.fori_loop(0, ZROWS, _z, 0)

    def _pass(c, _):
        for q in range(8):
            pltpu.sync_copy(zbuf, acc.at[pl.ds(sid * TILE_ROWS + q * ZROWS, ZROWS)])
        plsc.subcore_barrier()

        def _chunk(ch, _):
            rowb = wid * TBATCH + ch * CCHUNK
            pltpu.sync_copy(dstc_hbm.at[c, pl.ds(rowb, CCHUNK)], dbuf)
            pltpu.sync_copy(sidx_hbm.at[pl.ds(rowb, CCHUNK)], sbuf)
            cps = [pltpu.async_copy(enc_hbm.at[dbuf.at[j]], rows.at[j], gsem)
                   for j in range(CCHUNK)]
            for cp in cps:
                cp.wait()
            for j in range(CCHUNK):
                pltpu.sync_copy(rows.at[j], acc.at[sbuf.at[j]], add=True)
            return 0

        lax.fori_loop(0, NCHUNK, _chunk, 0)
        plsc.subcore_barrier()
        pltpu.sync_copy(acc.at[pl.ds(sid * TILE_ROWS, TILE_ROWS)],
                        acc_out.at[cid, c, pl.ds(sid * TILE_ROWS, TILE_ROWS)])
        return 0

    lax.fori_loop(0, NPASS, _pass, 0)


_sc_phase1 = functools.partial(
    pl.kernel,
    out_type=[
        jax.ShapeDtypeStruct((NC, NPASS, ACC_ROWS, CHUNK), jnp.float32),
        jax.ShapeDtypeStruct((B, D), jnp.float32),
    ],
    mesh=plsc.VectorSubcoreMesh(core_axis_name="c", subcore_axis_name="s"),
    scratch_types=[
        pltpu.VMEM_SHARED((ACC_ROWS, CHUNK), jnp.float32),
        pltpu.VMEM((CCHUNK, EBATCH), jnp.int32),
        pltpu.VMEM((CCHUNK, EBATCH), jnp.int32),
        pltpu.VMEM((CCHUNK, EBATCH, CHUNK), jnp.float32),
        pltpu.VMEM((ZROWS, CHUNK), jnp.float32),
        pltpu.VMEM((B // NW,), jnp.int32),
        pltpu.VMEM((B // NW, D), jnp.float32),
        pltpu.SemaphoreType.DMA,
    ],
    compiler_params=pltpu.CompilerParams(use_tc_tiling_on_sc=False),
)(_sc_body)


def _red_body(acc_ref, en_ref, out_ref):
    s = pl.program_id(2)
    z = acc_ref[0, 0]
    e = en_ref[...]
    prod = lax.dot_general(z, e, (((0,), (0,)), ((), ())),
                           preferred_element_type=jnp.float32)

    @pl.when(s == 0)
    def _():
        out_ref[...] = prod[None, None]

    @pl.when(s != 0)
    def _():
        out_ref[...] += prod[None, None]


def _reduce(acc_all, en):
    return pl.pallas_call(
        _red_body,
        grid=(R, NPASS, NC),
        in_specs=[
            pl.BlockSpec((1, 1, N, CHUNK), lambda r, c, s: (s, c, r, 0)),
            pl.BlockSpec((N, D), lambda r, c, s: (0, 0)),
        ],
        out_specs=pl.BlockSpec((1, 1, CHUNK, D), lambda r, c, s: (r, c, 0, 0)),
        out_shape=jax.ShapeDtypeStruct((R, NPASS, CHUNK, D), jnp.float32),
    )(acc_all, en)


def _pred_body(h_ref, ridx_ref, rall_ref, out_ref):
    h = h_ref[...]
    ridx = ridx_ref[...]
    acc = jnp.zeros((B, D), jnp.float32)
    for r in range(R):
        pr = jnp.dot(h, rall_ref[r], preferred_element_type=jnp.float32)
        acc = acc + jnp.where(ridx == r, pr, 0.0)
    n = jnp.sqrt(jnp.sum(acc * acc, axis=1, keepdims=True))
    out_ref[...] = acc / jnp.maximum(n, EPS)


def _predict(hrows, ridx2, rall):
    return pl.pallas_call(
        _pred_body,
        in_specs=[
            pl.BlockSpec(memory_space=pltpu.VMEM),
            pl.BlockSpec(memory_space=pltpu.VMEM),
            pl.BlockSpec(memory_space=pltpu.VMEM),
        ],
        out_specs=pl.BlockSpec(memory_space=pltpu.VMEM),
        out_shape=jax.ShapeDtypeStruct((B, D), jnp.float32),
    )(hrows, ridx2, rall)


def kernel(h_idx, r_idx, edge_index, edge_rel, entity_emb):
    src = edge_index[0].astype(jnp.int32)
    dst = edge_index[1].astype(jnp.int32)
    rel = edge_rel.astype(jnp.int32)

    en = _normalize(entity_emb)
    enc = jnp.concatenate(
        [en.reshape(N, NPASS, CHUNK).transpose(1, 0, 2).reshape(NPASS * N, CHUNK),
         jnp.zeros((ENC_ROWS - NPASS * N, CHUNK), jnp.float32)])

    sidx2, dstc = _make_indices(src.reshape(E // 128, 128),
                                rel.reshape(E // 128, 128),
                                dst.reshape(E // 128, 128))

    npad = EP - E
    pad_g = jnp.full((NPASS, npad), SROWS, jnp.int32)
    sidxp = jnp.concatenate(
        [sidx2.reshape(-1), jnp.zeros((npad,), jnp.int32)]).reshape(EROWS, EBATCH)
    dstcp = jnp.concatenate(
        [dstc.reshape(NPASS, E), pad_g], axis=1).reshape(NPASS, EROWS, EBATCH)

    acc_all, hrows = _sc_phase1(enc, dstcp, sidxp, h_idx.astype(jnp.int32), en)

    rblk = _reduce(acc_all, en)
    rall = rblk.transpose(0, 3, 1, 2).reshape(R, D, D)

    return _predict(hrows, r_idx.astype(jnp.int32).reshape(B, 1), rall)
